# baseline (speedup 1.0000x reference)
import functools
import math

import numpy as np
import jax
import jax.numpy as jnp
from jax import lax
from jax.experimental import pallas as pl
from jax.experimental.pallas import tpu as pltpu
from jax.experimental.pallas import tpu_sc as plsc

_N = 10000
_E = 320000
_D = 128
_B = 64
_K = 8
_T = 1.0

_NP = 10240
_HALF = _NP // 2
_PADR = 128
_NCORES = 2
_NSUB = 16
_NTILES = _NCORES * _NSUB
_RPT = _HALF // _NSUB
_C = 128
_NCH = 84
_EPC = _NCH * _C * _NSUB
_BLK = 2048

_CK = [(-1.0) ** k * (_T ** k) / math.factorial(k) for k in range(_K + 1)]
_DJ = [((-1.0) ** j) * sum(_CK[k] * math.comb(k, j) for k in range(j, _K + 1))
       for j in range(_K + 1)]


def _sc_mesh():
    return plsc.VectorSubcoreMesh(
        core_axis_name="c", subcore_axis_name="s",
        num_cores=_NCORES, num_subcores=_NSUB)


def _sc_step(src3, dst3, term, rec):
    @functools.partial(
        pl.kernel,
        out_type=jax.ShapeDtypeStruct((_NP, _D), jnp.float32),
        mesh=_sc_mesh(),
        scratch_types=[
            pltpu.VMEM((_NCH, _C), jnp.int32),
            pltpu.VMEM((_NCH, _C), jnp.int32),
            pltpu.VMEM((_C, _D), jnp.float32),
            pltpu.VMEM((_C, _D), jnp.float32),
            pltpu.VMEM_SHARED((_HALF + _PADR, _D), jnp.float32),
            pltpu.SemaphoreType.DMA,
            pltpu.SemaphoreType.DMA,
        ],
    )
    def k(src_h, dst_h, term_h, rec_h, out_h, sidx, didx, rows0, rows1,
          agg_sh, sem0, sem1):
        c = lax.axis_index("c")
        s = lax.axis_index("s")
        wid = c * _NSUB + s

        def zrow(r, carry):
            for j in range(_D // 16):
                rows1[r, pl.ds(j * 16, 16)] = jnp.zeros((16,), jnp.float32)
            return carry
        lax.fori_loop(0, _C, zrow, 0)

        base = s * _RPT
        for off, sz in ((0, 128), (128, 128), (256, 64)):
            pltpu.sync_copy(rows1.at[pl.ds(0, sz)],
                            agg_sh.at[pl.ds(base + off, sz)])
        @pl.when(s == 0)
        def _():
            pltpu.sync_copy(rows1, agg_sh.at[pl.ds(_HALF, _PADR)])

        pltpu.sync_copy(src_h.at[wid], sidx)
        pltpu.sync_copy(dst_h.at[wid], didx)
        plsc.subcore_barrier()

        nhalf = _NCH // 2
        pltpu.async_copy(term_h.at[sidx.at[0]], rows0, sem0)

        def step2(g2, carry):
            g = g2 * 2
            pltpu.make_async_copy(term_h.at[sidx.at[g]], rows0, sem0).wait()
            pltpu.async_copy(term_h.at[sidx.at[g + 1]], rows1, sem1)
            pltpu.sync_copy(rows0, agg_sh.at[didx.at[g]], add=True)
            pltpu.make_async_copy(term_h.at[sidx.at[g + 1]], rows1,
                                  sem1).wait()

            @pl.when(g2 + 1 < nhalf)
            def _():
                pltpu.async_copy(term_h.at[sidx.at[g + 2]], rows0, sem0)

            pltpu.sync_copy(rows1, agg_sh.at[didx.at[g + 1]], add=True)
            return carry
        lax.fori_loop(0, nhalf, step2, 0)
        plsc.subcore_barrier()

        gbase = c * _HALF + base
        for off, sz in ((0, 128), (128, 128), (256, 64)):
            pltpu.sync_copy(agg_sh.at[pl.ds(base + off, sz)],
                            rows0.at[pl.ds(0, sz)])
            pltpu.sync_copy(rec_h.at[pl.ds(gbase + off, sz)],
                            rows1.at[pl.ds(0, sz)])

            def mul(r, carry):
                for j in range(_D // 16):
                    rows0[r, pl.ds(j * 16, 16)] = (
                        rows0[r, pl.ds(j * 16, 16)]
                        * rows1[r, pl.ds(j * 16, 16)])
                return carry
            lax.fori_loop(0, sz, mul, 0)
            pltpu.sync_copy(rows0.at[pl.ds(0, sz)],
                            out_h.at[pl.ds(gbase + off, sz)])

    return k(src3, dst3, term, rec)


_SQRT1_2 = 0.7071067811865476


def _gelu(x):
    return 0.5 * x * (1.0 + lax.erf(x * _SQRT1_2))


def _full(shape):
    return pl.BlockSpec(shape, lambda i: (0,) * len(shape))


def _rows(d):
    return pl.BlockSpec((_BLK, d), lambda i: (i, 0))


def _tc_linear(xt, W, b):
    dout, din = W.shape

    def body(x_ref, w_ref, b_ref, o_ref):
        o_ref[...] = (
            jnp.dot(x_ref[...], w_ref[...].T, preferred_element_type=jnp.float32)
            + b_ref[...])

    return pl.pallas_call(
        body,
        grid=(_NP // _BLK,),
        in_specs=[_rows(din), _full((dout, din)), _full((1, dout))],
        out_specs=_rows(dout),
        out_shape=jax.ShapeDtypeStruct((_NP, dout), jnp.float32),
    )(xt, W, b)


def _tc_prologue(ht, w1, b1, w2, b2, wl, bl):
    def body(h_ref, w1_ref, b1_ref, w2_ref, b2_ref, wl_ref, bl_ref,
             H_ref, c_ref, s_ref):
        hb = h_ref[...]
        a = _gelu(jnp.dot(hb, w1_ref[...].T, preferred_element_type=jnp.float32)
                  + b1_ref[...])
        ang = (jnp.dot(a, w2_ref[...].T, preferred_element_type=jnp.float32)
               + b2_ref[...])
        cv = jnp.cos(ang)
        sv = jnp.sin(ang)
        X = hb[:, :_B]
        Y = hb[:, _B:]
        hr = jnp.concatenate([cv * X - sv * Y, sv * X + cv * Y], axis=1)
        H_ref[...] = (jnp.dot(hr, wl_ref[...].T,
                              preferred_element_type=jnp.float32) + bl_ref[...])
        c_ref[...] = cv
        s_ref[...] = sv

    return pl.pallas_call(
        body,
        grid=(_NP // _BLK,),
        in_specs=[_rows(_D), _full((_D, _D)), _full((1, _D)),
                  _full((_B, _D)), _full((1, _B)),
                  _full((_D, _D)), _full((1, _D))],
        out_specs=[_rows(_D), _rows(_B), _rows(_B)],
        out_shape=[jax.ShapeDtypeStruct((_NP, _D), jnp.float32),
                   jax.ShapeDtypeStruct((_NP, _B), jnp.float32),
                   jax.ShapeDtypeStruct((_NP, _B), jnp.float32)],
    )(ht, w1, b1, w2, b2, wl, bl)


def _tc_recip(deg):
    def body(d_ref, o_ref):
        o_ref[...] = 1.0 / jnp.maximum(d_ref[...], 1.0)

    return pl.pallas_call(
        body,
        grid=(_NP // _BLK,),
        in_specs=[_rows(_D)],
        out_specs=_rows(_D),
        out_shape=jax.ShapeDtypeStruct((_NP, _D), jnp.float32),
    )(deg)


def _tc_epilogue(us, cv, sv, ht):
    def body(u0, u1, u2, u3, u4, u5, u6, u7, u8, c_ref, s_ref, h_ref, o_ref):
        urefs = (u0, u1, u2, u3, u4, u5, u6, u7, u8)
        acc = _DJ[0] * u0[...]
        for j in range(1, _K + 1):
            acc = acc + _DJ[j] * urefs[j][...]
        cb = c_ref[...]
        sb = s_ref[...]
        Xr = acc[:, :_B]
        Yr = acc[:, _B:]
        ho = jnp.concatenate([cb * Xr + sb * Yr, -sb * Xr + cb * Yr], axis=1)
        o_ref[...] = h_ref[...] + _gelu(ho)

    return pl.pallas_call(
        body,
        grid=(_NP // _BLK,),
        in_specs=[_rows(_D)] * (_K + 1) + [_rows(_B), _rows(_B), _rows(_D)],
        out_specs=_rows(_D),
        out_shape=jax.ShapeDtypeStruct((_NP, _D), jnp.float32),
    )(*us, cv, sv, ht)


_PERM = np.concatenate([np.arange(0, _D, 2), np.arange(1, _D, 2)])


def _route_edges(src, dst):
    half = (dst >= _HALF).astype(jnp.int32)
    cum1 = jnp.cumsum(half)
    pos = jnp.where(half == 1,
                    _EPC + cum1 - 1,
                    jnp.arange(_E, dtype=jnp.int32) - cum1)
    srcb = jnp.zeros((2 * _EPC,), jnp.int32).at[pos].set(src, mode="drop")
    dflt = _HALF + (jnp.arange(2 * _EPC, dtype=jnp.int32) % _PADR)
    dstb = dflt.at[pos].set(dst - half * _HALF, mode="drop")

    def to_tiles(b):
        b = b.reshape(_NCORES, _NCH * _NSUB, _C)
        b = b.reshape(_NCORES, _NCH, _NSUB, _C).transpose(0, 2, 1, 3)
        return b.reshape(_NTILES, _NCH, _C)

    return to_tiles(srcb), to_tiles(dstb)


def kernel(x, edge_index, W_in, b_in,
           phi_w1_0, phi_b1_0, phi_w2_0, phi_b2_0,
           phi_w1_1, phi_b1_1, phi_w2_1, phi_b2_1,
           lt_w_0, lt_b_0, lt_w_1, lt_b_1,
           W_out, b_out):
    p = _PERM
    W_in2 = W_in[p, :]
    b_in2 = b_in[p][None]
    phis = [(phi_w1_0[:, p], phi_b1_0[None], phi_w2_0, phi_b2_0[None]),
            (phi_w1_1[:, p], phi_b1_1[None], phi_w2_1, phi_b2_1[None])]
    lts = [(lt_w_0[p][:, p], lt_b_0[p][None]),
           (lt_w_1[p][:, p], lt_b_1[p][None])]
    W_out2 = W_out[:, p]

    xp = jnp.pad(x, ((0, _NP - _N), (0, 0)))
    src3, dst3 = _route_edges(edge_index[0], edge_index[1])

    ones = jnp.ones((_NP, _D), jnp.float32)
    deg = _sc_step(src3, dst3, ones, ones)
    rec = _tc_recip(deg)

    ht = _tc_linear(xp, W_in2, b_in2)
    for l in range(2):
        w1, b1, w2, b2 = phis[l]
        wl, bl = lts[l]
        Ht, cv, sv = _tc_prologue(ht, w1, b1, w2, b2, wl, bl)
        us = [Ht]
        u = Ht
        for _ in range(_K):
            u = _sc_step(src3, dst3, u, rec)
            us.append(u)
        ht = _tc_epilogue(us, cv, sv, ht)

    out = _tc_linear(ht, W_out2, b_out[None])
    return out[:_N]
